# baseline (speedup 1.0000x reference)
import functools

import jax
import jax.numpy as jnp
import numpy as np
from jax import lax
from jax.experimental import pallas as pl
from jax.experimental.pallas import tpu as pltpu
from jax.experimental.pallas import tpu_sc as plsc

NUM_GENES = 2000
DH = 256
DU = 1024
NE = 4
BATCH = 8192
BB = 512
NBB = BATCH // BB

BLK = 256
NBLK = BATCH // BLK + NE
NSLOT = NBLK * BLK

NW = 32
CHUNK = BATCH // NW
NV = CHUNK // 16


def _gelu(x):
    return 0.5 * x * (1.0 + jax.lax.erf(x * np.float32(1.0 / np.sqrt(2.0))))


def _enc_body(vis, pos, grad, B_f, pos_w, pos_b, img_w, img_b, rw, rb,
              z_out, eid_out, gate_out, cnt_out):
    xp = jnp.float32(2.0 * np.pi) * jnp.dot(pos[...], B_f[...],
                                            preferred_element_type=jnp.float32)
    four = jnp.concatenate([jnp.sin(xp), jnp.cos(xp)], axis=-1)
    penc = _gelu(jnp.dot(four, pos_w[...], preferred_element_type=jnp.float32)
                 + pos_b[...])
    z = jnp.dot(vis[...], img_w[...], preferred_element_type=jnp.float32) \
        + img_b[...] + penc
    z_out[...] = z
    rw_z = rw[0:DH, :]
    rw_g = rw[DH:DH + 1, :]
    logits = jnp.dot(z, rw_z, preferred_element_type=jnp.float32) \
        + grad[...] * rw_g + rb[...]
    m = jnp.max(logits, axis=-1, keepdims=True)
    e = jnp.exp(logits - m)
    probs = e / jnp.sum(e, axis=-1, keepdims=True)
    pmax = jnp.max(probs, axis=-1, keepdims=True)
    ids = jax.lax.broadcasted_iota(jnp.int32, probs.shape, 1)
    eid = jnp.min(jnp.where(probs >= pmax, ids, NE), axis=-1, keepdims=True)
    eid_out[...] = eid
    gate_out[...] = pmax
    onehot = (ids == eid).astype(jnp.int32)
    rows = jax.lax.broadcasted_iota(jnp.int32, onehot.shape, 0)
    c_lo = jnp.sum(jnp.where(rows < BB // 2, onehot, 0), axis=0,
                   keepdims=True)
    c_hi = jnp.sum(jnp.where(rows >= BB // 2, onehot, 0), axis=0,
                   keepdims=True)
    cnt_out[...] = jnp.concatenate([c_lo, c_hi], axis=0)[None]


def _encoder(vis, pos, grad, B_f, pos_w, pos_b, img_w, img_b, rw, rb):
    return pl.pallas_call(
        _enc_body,
        grid=(NBB,),
        in_specs=[
            pl.BlockSpec((BB, DU), lambda i: (i, 0)),
            pl.BlockSpec((BB, 3), lambda i: (i, 0)),
            pl.BlockSpec((BB, 1), lambda i: (i, 0)),
            pl.BlockSpec((3, 128), lambda i: (0, 0)),
            pl.BlockSpec((DH, DH), lambda i: (0, 0)),
            pl.BlockSpec((DH,), lambda i: (0,)),
            pl.BlockSpec((DU, DH), lambda i: (0, 0)),
            pl.BlockSpec((DH,), lambda i: (0,)),
            pl.BlockSpec((DH + 1, NE), lambda i: (0, 0)),
            pl.BlockSpec((NE,), lambda i: (0,)),
        ],
        out_specs=[
            pl.BlockSpec((BB, DH), lambda i: (i, 0)),
            pl.BlockSpec((BB, 1), lambda i: (i, 0)),
            pl.BlockSpec((BB, 1), lambda i: (i, 0)),
            pl.BlockSpec((1, 2, NE), lambda i: (i, 0, 0)),
        ],
        out_shape=[
            jax.ShapeDtypeStruct((BATCH, DH), jnp.float32),
            jax.ShapeDtypeStruct((BATCH, 1), jnp.int32),
            jax.ShapeDtypeStruct((BATCH, 1), jnp.float32),
            jax.ShapeDtypeStruct((NBB, 2, NE), jnp.int32),
        ],
        compiler_params=pltpu.CompilerParams(
            dimension_semantics=("arbitrary",)),
    )(vis, pos, grad, B_f, pos_w, pos_b, img_w, img_b, rw, rb)


def _route_dests(cnt_v, eid_v, dest_v, wid):
    iota16 = lax.iota(jnp.int32, 16)
    tot = [jnp.int32(0)] * NE
    pri = [jnp.int32(0)] * NE
    for r in range(NW * NE // 16):
        cv = cnt_v[pl.ds(r * 16, 16)]
        idx = r * 16 + iota16
        w_of = lax.shift_right_logical(idx, 2)
        e_of = lax.bitwise_and(idx, 3)
        before = w_of < wid
        for e in range(NE):
            me = e_of == e
            tot[e] = tot[e] + jnp.sum(jnp.where(me, cv, 0))
            pri[e] = pri[e] + jnp.sum(jnp.where(me & before, cv, 0))

    pad = [((c + BLK - 1) // BLK) * BLK for c in tot]
    s0 = jnp.int32(0)
    s1 = pad[0]
    s2 = pad[0] + pad[1]
    s3 = pad[0] + pad[1] + pad[2]
    starts = (s0, s1, s2, s3)
    base = tuple(starts[e] + pri[e] for e in range(NE))

    run = [jnp.int32(0)] * NE
    for j in range(2):
        for k in range(NV // 2):
            off = j * 128 + k * 16
            ev = eid_v[pl.ds(off, 16)]
            dest = jnp.zeros((16,), jnp.int32)
            for e in range(NE):
                m = ev == e
                cs = plsc.cumsum(jnp.where(m, 1, 0).astype(jnp.int32))
                dest = jnp.where(m, base[e] + run[e] + cs - 1, dest)
                run[e] = run[e] + jnp.max(cs)
            dest_v[j, pl.ds(k * 16, 16)] = dest
    return starts


def _dispatch_body(cnt_hbm, eid_hbm, z_hbm, zs_out, be_out,
                   cnt_v, eid_v, dest_v, zstage0, zstage1, blk_v,
                   sem1, semg):
    wid = lax.axis_index("s") * 2 + lax.axis_index("c")

    pltpu.sync_copy(cnt_hbm, cnt_v)
    pltpu.sync_copy(eid_hbm.at[pl.ds(wid * CHUNK, CHUNK)], eid_v)
    g0 = pltpu.async_copy(z_hbm.at[pl.ds(wid * CHUNK, 128)], zstage0, semg)
    g1 = pltpu.async_copy(z_hbm.at[pl.ds(wid * CHUNK + 128, 128)], zstage1,
                          semg)

    starts = _route_dests(cnt_v, eid_v, dest_v, wid)
    s1, s2, s3 = starts[1], starts[2], starts[3]

    g0.wait()
    put0 = pltpu.async_copy(zstage0, zs_out.at[dest_v.at[0]], sem1)
    g1.wait()
    put1 = pltpu.async_copy(zstage1, zs_out.at[dest_v.at[1]], sem1)

    @pl.when(wid == 0)
    def _():
        iota16 = lax.iota(jnp.int32, 16)
        for k in range(3):
            b = (k * 16 + iota16) * BLK
            be = (b >= s1).astype(jnp.int32) + (b >= s2).astype(jnp.int32) \
                + (b >= s3).astype(jnp.int32)
            blk_v[pl.ds(k * 16, 16)] = be
        pltpu.sync_copy(blk_v, be_out)

    put0.wait()
    put1.wait()


def _dispatch(cnt, eid, z):
    mesh = plsc.VectorSubcoreMesh(core_axis_name="c", subcore_axis_name="s")
    f = functools.partial(
        pl.kernel,
        mesh=mesh,
        out_type=[
            jax.ShapeDtypeStruct((NSLOT, DH), jnp.float32),
            jax.ShapeDtypeStruct((48,), jnp.int32),
        ],
        scratch_types=[
            pltpu.VMEM((NW * NE,), jnp.int32),
            pltpu.VMEM((CHUNK,), jnp.int32),
            pltpu.VMEM((2, 128), jnp.int32),
            pltpu.VMEM((128, DH), jnp.float32),
            pltpu.VMEM((128, DH), jnp.float32),
            pltpu.VMEM((48,), jnp.int32),
            pltpu.SemaphoreType.DMA,
            pltpu.SemaphoreType.DMA,
        ],
        compiler_params=pltpu.CompilerParams(needs_layout_passes=False),
    )(_dispatch_body)
    return f(cnt, eid, z)


def _moe_sp_body(be_ref, z_ref, w1, b1, w2, b2, out):
    del be_ref
    zb = z_ref[...].astype(jnp.bfloat16)
    h = _gelu(jnp.dot(zb, w1[0], preferred_element_type=jnp.float32)
              + b1[0, 0])
    out[...] = jnp.dot(h.astype(jnp.bfloat16), w2[0],
                       preferred_element_type=jnp.float32) + b2[0, 0]


def _moe_grouped(zsorted, blkexp, ew1, eb1, ew2, eb2):
    return pl.pallas_call(
        _moe_sp_body,
        grid_spec=pltpu.PrefetchScalarGridSpec(
            num_scalar_prefetch=1,
            grid=(NBLK,),
            in_specs=[
                pl.BlockSpec((BLK, DH), lambda i, be: (i, 0)),
                pl.BlockSpec((1, DH, 4 * DH), lambda i, be: (be[i], 0, 0)),
                pl.BlockSpec((1, 1, 4 * DH), lambda i, be: (be[i], 0, 0)),
                pl.BlockSpec((1, 4 * DH, DH), lambda i, be: (be[i], 0, 0)),
                pl.BlockSpec((1, 1, DH), lambda i, be: (be[i], 0, 0)),
            ],
            out_specs=pl.BlockSpec((BLK, DH), lambda i, be: (i, 0)),
        ),
        out_shape=jax.ShapeDtypeStruct((NSLOT, DH), jnp.float32),
        compiler_params=pltpu.CompilerParams(
            dimension_semantics=("arbitrary",)),
    )(blkexp, zsorted, ew1, eb1.reshape(NE, 1, 4 * DH), ew2,
      eb2.reshape(NE, 1, DH))


def _unsort_body(cnt_hbm, eid_hbm, h_hbm, moe_out,
                 cnt_v, eid_v, dest_v, h0, h1, semg, sems):
    wid = lax.axis_index("s") * 2 + lax.axis_index("c")

    pltpu.sync_copy(cnt_hbm, cnt_v)
    pltpu.sync_copy(eid_hbm.at[pl.ds(wid * CHUNK, CHUNK)], eid_v)
    _route_dests(cnt_v, eid_v, dest_v, wid)

    g0 = pltpu.async_copy(h_hbm.at[dest_v.at[0]], h0, semg)
    g1 = pltpu.async_copy(h_hbm.at[dest_v.at[1]], h1, semg)
    g0.wait()
    p0 = pltpu.async_copy(h0, moe_out.at[pl.ds(wid * CHUNK, 128)], sems)
    g1.wait()
    p1 = pltpu.async_copy(h1, moe_out.at[pl.ds(wid * CHUNK + 128, 128)],
                          sems)
    p0.wait()
    p1.wait()


def _unsort(cnt, eid, hsorted):
    mesh = plsc.VectorSubcoreMesh(core_axis_name="c", subcore_axis_name="s")
    f = functools.partial(
        pl.kernel,
        mesh=mesh,
        out_type=jax.ShapeDtypeStruct((BATCH, DH), jnp.float32),
        scratch_types=[
            pltpu.VMEM((NW * NE,), jnp.int32),
            pltpu.VMEM((CHUNK,), jnp.int32),
            pltpu.VMEM((2, 128), jnp.int32),
            pltpu.VMEM((128, DH), jnp.float32),
            pltpu.VMEM((128, DH), jnp.float32),
            pltpu.SemaphoreType.DMA,
            pltpu.SemaphoreType.DMA,
        ],
        compiler_params=pltpu.CompilerParams(needs_layout_passes=False),
    )(_unsort_body)
    return f(cnt, eid, hsorted)


def _dec_body(z, moe, gate, dw1, db1, ln_g, ln_b, w_mu, b_mu, w_th, b_th,
              aw1, ab1, aw2, ab2, fw1, fb1, fw2, fb2,
              mu_out, th_out, fn_out, al_out):
    z2 = z[...] + gate[...] * moe[...]
    h = jnp.dot(z2, dw1[...], preferred_element_type=jnp.float32) + db1[...]
    m = jnp.mean(h, axis=-1, keepdims=True)
    hc = h - m
    v = jnp.mean(hc * hc, axis=-1, keepdims=True)
    h = hc * jax.lax.rsqrt(v + 1e-5) * ln_g[...] + ln_b[...]
    h = _gelu(h)
    h16 = h.astype(jnp.bfloat16)
    mu_lin = jnp.dot(h16, w_mu[...], preferred_element_type=jnp.float32) + b_mu[...]
    th_lin = jnp.dot(h16, w_th[...], preferred_element_type=jnp.float32) + b_th[...]
    sp = lambda x: jnp.maximum(x, 0.0) + jnp.log(1.0 + jnp.exp(-jnp.abs(x)))
    zpad = jnp.zeros((mu_lin.shape[0], 2048 - NUM_GENES), jnp.float32)
    mu_out[...] = jnp.concatenate([sp(mu_lin), zpad], axis=1)
    th_out[...] = jnp.concatenate([sp(th_lin), zpad], axis=1)
    fh = _gelu(jnp.dot(z2, fw1[...], preferred_element_type=jnp.float32) + fb1[...])
    fn = jnp.dot(fh, fw2[...], preferred_element_type=jnp.float32) + fb2[...]
    fn_out[...] = jax.nn.sigmoid(fn)
    ah = _gelu(jnp.dot(z2, aw1[...], preferred_element_type=jnp.float32) + ab1[...])
    al_out[...] = jnp.dot(ah, aw2[...], preferred_element_type=jnp.float32) + ab2[...]


def _decoder(z, moe, gate, dw1, db1, ln_g, ln_b, w_mu, b_mu, w_th, b_th,
             aw1, ab1, aw2, ab2, fw1, fb1, fw2, fb2):
    full = lambda *shape: pl.BlockSpec(shape, lambda i: (0,) * len(shape))
    row = lambda *shape: pl.BlockSpec(shape, lambda i: (i,) + (0,) * (len(shape) - 1))
    return pl.pallas_call(
        _dec_body,
        grid=(NBB,),
        in_specs=[
            row(BB, DH), row(BB, DH), row(BB, 1),
            full(DH, DH), full(DH), full(DH), full(DH),
            full(DH, NUM_GENES), full(NUM_GENES),
            full(DH, NUM_GENES), full(NUM_GENES),
            full(DH, 128), full(128), full(128, 30), full(30),
            full(DH, 64), full(64), full(64, 1), full(1),
        ],
        out_specs=[
            row(BB, 2048), row(BB, 2048), row(BB, 1), row(BB, 30),
        ],
        out_shape=[
            jax.ShapeDtypeStruct((BATCH, 2048), jnp.float32),
            jax.ShapeDtypeStruct((BATCH, 2048), jnp.float32),
            jax.ShapeDtypeStruct((BATCH, 1), jnp.float32),
            jax.ShapeDtypeStruct((BATCH, 30), jnp.float32),
        ],
        compiler_params=pltpu.CompilerParams(
            dimension_semantics=("arbitrary",)),
    )(z, moe, gate, dw1, db1, ln_g, ln_b, w_mu, b_mu, w_th, b_th,
      aw1, ab1, aw2, ab2, fw1, fb1, fw2, fb2)


def kernel(vis, pos, grad, lib, B_f, pos_w, pos_b, img_w, img_b, router_w,
           router_b, ew1, eb1, ew2, eb2, dec_w1, dec_b1, ln_g, ln_b, dec_w2,
           dec_b2, al_w1, al_b1, al_w2, al_b2, fn_w1, fn_b1, fn_w2, fn_b2):
    z, eid, gate, cnt = _encoder(vis, pos, grad, B_f, pos_w, pos_b, img_w,
                                 img_b, router_w, router_b)
    cnt_flat = cnt.reshape(NW * NE)
    eid_flat = eid.reshape(BATCH)
    zsorted, blkexp = _dispatch(cnt_flat, eid_flat, z)
    hsorted = _moe_grouped(zsorted, blkexp, ew1.astype(jnp.bfloat16), eb1,
                           ew2.astype(jnp.bfloat16), eb2)
    moe = _unsort(cnt_flat, eid_flat, hsorted)
    w_mu = dec_w2[:, 0::2].astype(jnp.bfloat16)
    w_th = dec_w2[:, 1::2].astype(jnp.bfloat16)
    b_mu = dec_b2[0::2]
    b_th = dec_b2[1::2]
    spmu, spth, func, align = _decoder(
        z, moe, gate, dec_w1, dec_b1, ln_g, ln_b, w_mu, b_mu, w_th, b_th,
        al_w1, al_b1, al_w2, al_b2, fn_w1, fn_b1, fn_w2, fn_b2)
    mu = spmu[:, :NUM_GENES] * lib + 1e-06
    theta = spth[:, :NUM_GENES] + 1e-06
    return (mu, theta, func, align)


# speedup vs baseline: 1.2101x; 1.2101x over previous
import functools

import jax
import jax.numpy as jnp
import numpy as np
from jax import lax
from jax.experimental import pallas as pl
from jax.experimental.pallas import tpu as pltpu
from jax.experimental.pallas import tpu_sc as plsc

NUM_GENES = 2000
DH = 256
DU = 1024
NE = 4
BATCH = 8192
BB = 512
NBB = BATCH // BB

BLK = 256
NBLK = BATCH // BLK + NE
NSLOT = NBLK * BLK

NW = 32
CHUNK = BATCH // NW
NV = CHUNK // 16


def _gelu(x):
    return 0.5 * x * (1.0 + jax.lax.erf(x * np.float32(1.0 / np.sqrt(2.0))))


def _enc_body(vis, pos, grad, B_f, pos_w, pos_b, img_w, img_b, rw, rb,
              z_out, eid_out, gate_out, cnt_out):
    xp = jnp.float32(2.0 * np.pi) * jnp.dot(pos[...], B_f[...],
                                            preferred_element_type=jnp.float32)
    four = jnp.concatenate([jnp.sin(xp), jnp.cos(xp)], axis=-1)
    penc = _gelu(jnp.dot(four, pos_w[...], preferred_element_type=jnp.float32)
                 + pos_b[...])
    z = jnp.dot(vis[...], img_w[...], preferred_element_type=jnp.float32) \
        + img_b[...] + penc
    z_out[...] = z
    rw_z = rw[0:DH, :]
    rw_g = rw[DH:DH + 1, :]
    logits = jnp.dot(z, rw_z, preferred_element_type=jnp.float32) \
        + grad[...] * rw_g + rb[...]
    m = jnp.max(logits, axis=-1, keepdims=True)
    e = jnp.exp(logits - m)
    probs = e / jnp.sum(e, axis=-1, keepdims=True)
    pmax = jnp.max(probs, axis=-1, keepdims=True)
    ids = jax.lax.broadcasted_iota(jnp.int32, probs.shape, 1)
    eid = jnp.min(jnp.where(probs >= pmax, ids, NE), axis=-1, keepdims=True)
    eid_out[...] = eid
    gate_out[...] = pmax
    onehot = (ids == eid).astype(jnp.int32)
    rows = jax.lax.broadcasted_iota(jnp.int32, onehot.shape, 0)
    c_lo = jnp.sum(jnp.where(rows < BB // 2, onehot, 0), axis=0,
                   keepdims=True)
    c_hi = jnp.sum(jnp.where(rows >= BB // 2, onehot, 0), axis=0,
                   keepdims=True)
    cnt_out[...] = jnp.concatenate([c_lo, c_hi], axis=0)[None]


def _encoder(vis, pos, grad, B_f, pos_w, pos_b, img_w, img_b, rw, rb):
    return pl.pallas_call(
        _enc_body,
        grid=(NBB,),
        in_specs=[
            pl.BlockSpec((BB, DU), lambda i: (i, 0)),
            pl.BlockSpec((BB, 3), lambda i: (i, 0)),
            pl.BlockSpec((BB, 1), lambda i: (i, 0)),
            pl.BlockSpec((3, 128), lambda i: (0, 0)),
            pl.BlockSpec((DH, DH), lambda i: (0, 0)),
            pl.BlockSpec((DH,), lambda i: (0,)),
            pl.BlockSpec((DU, DH), lambda i: (0, 0)),
            pl.BlockSpec((DH,), lambda i: (0,)),
            pl.BlockSpec((DH + 1, NE), lambda i: (0, 0)),
            pl.BlockSpec((NE,), lambda i: (0,)),
        ],
        out_specs=[
            pl.BlockSpec((BB, DH), lambda i: (i, 0)),
            pl.BlockSpec((BB, 1), lambda i: (i, 0)),
            pl.BlockSpec((BB, 1), lambda i: (i, 0)),
            pl.BlockSpec((1, 2, NE), lambda i: (i, 0, 0)),
        ],
        out_shape=[
            jax.ShapeDtypeStruct((BATCH, DH), jnp.float32),
            jax.ShapeDtypeStruct((BATCH, 1), jnp.int32),
            jax.ShapeDtypeStruct((BATCH, 1), jnp.float32),
            jax.ShapeDtypeStruct((NBB, 2, NE), jnp.int32),
        ],
        compiler_params=pltpu.CompilerParams(
            dimension_semantics=("arbitrary",)),
    )(vis, pos, grad, B_f, pos_w, pos_b, img_w, img_b, rw, rb)


def _route_dests(cnt_v, eid_v, dest_v, wid):
    iota16 = lax.iota(jnp.int32, 16)
    tot = [jnp.int32(0)] * NE
    pri = [jnp.int32(0)] * NE
    for r in range(NW * NE // 16):
        cv = cnt_v[pl.ds(r * 16, 16)]
        idx = r * 16 + iota16
        w_of = lax.shift_right_logical(idx, 2)
        e_of = lax.bitwise_and(idx, 3)
        before = w_of < wid
        for e in range(NE):
            me = e_of == e
            tot[e] = tot[e] + jnp.sum(jnp.where(me, cv, 0))
            pri[e] = pri[e] + jnp.sum(jnp.where(me & before, cv, 0))

    pad = [((c + BLK - 1) // BLK) * BLK for c in tot]
    s0 = jnp.int32(0)
    s1 = pad[0]
    s2 = pad[0] + pad[1]
    s3 = pad[0] + pad[1] + pad[2]
    starts = (s0, s1, s2, s3)
    base = tuple(starts[e] + pri[e] for e in range(NE))

    run = [jnp.int32(0)] * NE
    for j in range(2):
        for k in range(NV // 2):
            off = j * 128 + k * 16
            ev = eid_v[pl.ds(off, 16)]
            dest = jnp.zeros((16,), jnp.int32)
            for e in range(NE):
                m = ev == e
                cs = plsc.cumsum(jnp.where(m, 1, 0).astype(jnp.int32))
                dest = jnp.where(m, base[e] + run[e] + cs - 1, dest)
                run[e] = run[e] + jnp.max(cs)
            dest_v[j, pl.ds(k * 16, 16)] = dest
    return starts


def _dispatch_body(cnt_hbm, eid_hbm, z_hbm, zs_out, be_out,
                   cnt_v, eid_v, dest_v, zstage0, zstage1, blk_v,
                   sem1, semg):
    wid = lax.axis_index("s") * 2 + lax.axis_index("c")

    pltpu.sync_copy(cnt_hbm, cnt_v)
    pltpu.sync_copy(eid_hbm.at[pl.ds(wid * CHUNK, CHUNK)], eid_v)
    g0 = pltpu.async_copy(z_hbm.at[pl.ds(wid * CHUNK, 128)], zstage0, semg)
    g1 = pltpu.async_copy(z_hbm.at[pl.ds(wid * CHUNK + 128, 128)], zstage1,
                          semg)

    starts = _route_dests(cnt_v, eid_v, dest_v, wid)
    s1, s2, s3 = starts[1], starts[2], starts[3]

    g0.wait()
    put0 = pltpu.async_copy(zstage0, zs_out.at[dest_v.at[0]], sem1)
    g1.wait()
    put1 = pltpu.async_copy(zstage1, zs_out.at[dest_v.at[1]], sem1)

    @pl.when(wid == 0)
    def _():
        iota16 = lax.iota(jnp.int32, 16)
        for k in range(3):
            b = (k * 16 + iota16) * BLK
            be = (b >= s1).astype(jnp.int32) + (b >= s2).astype(jnp.int32) \
                + (b >= s3).astype(jnp.int32)
            blk_v[pl.ds(k * 16, 16)] = be
        pltpu.sync_copy(blk_v, be_out)

    put0.wait()
    put1.wait()


def _dispatch(cnt, eid, z):
    mesh = plsc.VectorSubcoreMesh(core_axis_name="c", subcore_axis_name="s")
    f = functools.partial(
        pl.kernel,
        mesh=mesh,
        out_type=[
            jax.ShapeDtypeStruct((NSLOT, DH), jnp.float32),
            jax.ShapeDtypeStruct((48,), jnp.int32),
        ],
        scratch_types=[
            pltpu.VMEM((NW * NE,), jnp.int32),
            pltpu.VMEM((CHUNK,), jnp.int32),
            pltpu.VMEM((2, 128), jnp.int32),
            pltpu.VMEM((128, DH), jnp.float32),
            pltpu.VMEM((128, DH), jnp.float32),
            pltpu.VMEM((48,), jnp.int32),
            pltpu.SemaphoreType.DMA,
            pltpu.SemaphoreType.DMA,
        ],
        compiler_params=pltpu.CompilerParams(needs_layout_passes=False),
    )(_dispatch_body)
    return f(cnt, eid, z)


def _moe_sp_body(be_ref, z_ref, w1, b1, w2, b2, out):
    del be_ref
    zb = z_ref[...].astype(jnp.bfloat16)
    h = _gelu(jnp.dot(zb, w1[0], preferred_element_type=jnp.float32)
              + b1[0, 0])
    out[...] = jnp.dot(h.astype(jnp.bfloat16), w2[0],
                       preferred_element_type=jnp.float32) + b2[0, 0]


def _moe_grouped(zsorted, blkexp, ew1, eb1, ew2, eb2):
    return pl.pallas_call(
        _moe_sp_body,
        grid_spec=pltpu.PrefetchScalarGridSpec(
            num_scalar_prefetch=1,
            grid=(NBLK,),
            in_specs=[
                pl.BlockSpec((BLK, DH), lambda i, be: (i, 0)),
                pl.BlockSpec((1, DH, 4 * DH), lambda i, be: (be[i], 0, 0)),
                pl.BlockSpec((1, 1, 4 * DH), lambda i, be: (be[i], 0, 0)),
                pl.BlockSpec((1, 4 * DH, DH), lambda i, be: (be[i], 0, 0)),
                pl.BlockSpec((1, 1, DH), lambda i, be: (be[i], 0, 0)),
            ],
            out_specs=pl.BlockSpec((BLK, DH), lambda i, be: (i, 0)),
        ),
        out_shape=jax.ShapeDtypeStruct((NSLOT, DH), jnp.float32),
        compiler_params=pltpu.CompilerParams(
            dimension_semantics=("arbitrary",)),
    )(blkexp, zsorted, ew1, eb1.reshape(NE, 1, 4 * DH), ew2,
      eb2.reshape(NE, 1, DH))


def _unsort_body(cnt_hbm, eid_hbm, h_hbm, moe_out,
                 cnt_v, eid_v, dest_v, h0, h1, semg, sems):
    wid = lax.axis_index("s") * 2 + lax.axis_index("c")

    pltpu.sync_copy(cnt_hbm, cnt_v)
    pltpu.sync_copy(eid_hbm.at[pl.ds(wid * CHUNK, CHUNK)], eid_v)
    _route_dests(cnt_v, eid_v, dest_v, wid)

    g0 = pltpu.async_copy(h_hbm.at[dest_v.at[0]], h0, semg)
    g1 = pltpu.async_copy(h_hbm.at[dest_v.at[1]], h1, semg)
    g0.wait()
    p0 = pltpu.async_copy(h0, moe_out.at[pl.ds(wid * CHUNK, 128)], sems)
    g1.wait()
    p1 = pltpu.async_copy(h1, moe_out.at[pl.ds(wid * CHUNK + 128, 128)],
                          sems)
    p0.wait()
    p1.wait()


def _unsort(cnt, eid, hsorted):
    mesh = plsc.VectorSubcoreMesh(core_axis_name="c", subcore_axis_name="s")
    f = functools.partial(
        pl.kernel,
        mesh=mesh,
        out_type=jax.ShapeDtypeStruct((BATCH, DH), jnp.float32),
        scratch_types=[
            pltpu.VMEM((NW * NE,), jnp.int32),
            pltpu.VMEM((CHUNK,), jnp.int32),
            pltpu.VMEM((2, 128), jnp.int32),
            pltpu.VMEM((128, DH), jnp.float32),
            pltpu.VMEM((128, DH), jnp.float32),
            pltpu.SemaphoreType.DMA,
            pltpu.SemaphoreType.DMA,
        ],
        compiler_params=pltpu.CompilerParams(needs_layout_passes=False),
    )(_unsort_body)
    return f(cnt, eid, hsorted)


def _dec_body(z, moe, gate, lib, dw1, db1, ln_g, ln_b, w_mu, b_mu, w_th, b_th,
              aw1, ab1, aw2, ab2, fw1, fb1, fw2, fb2,
              mu_out, th_out, fn_out, al_out):
    z2 = z[...] + gate[...] * moe[...]
    h = jnp.dot(z2, dw1[...], preferred_element_type=jnp.float32) + db1[...]
    m = jnp.mean(h, axis=-1, keepdims=True)
    hc = h - m
    v = jnp.mean(hc * hc, axis=-1, keepdims=True)
    h = hc * jax.lax.rsqrt(v + 1e-5) * ln_g[...] + ln_b[...]
    h = _gelu(h)
    h16 = h.astype(jnp.bfloat16)
    mu_lin = jnp.dot(h16, w_mu[...], preferred_element_type=jnp.float32) + b_mu[...]
    th_lin = jnp.dot(h16, w_th[...], preferred_element_type=jnp.float32) + b_th[...]
    sp = lambda x: jnp.maximum(x, 0.0) + jnp.log(1.0 + jnp.exp(-jnp.abs(x)))
    mu_out[...] = sp(mu_lin) * lib[...] + 1e-06
    th_out[...] = sp(th_lin) + 1e-06
    fh = _gelu(jnp.dot(z2, fw1[...], preferred_element_type=jnp.float32) + fb1[...])
    fn = jnp.dot(fh, fw2[...], preferred_element_type=jnp.float32) + fb2[...]
    fn_out[...] = jax.nn.sigmoid(fn)
    ah = _gelu(jnp.dot(z2, aw1[...], preferred_element_type=jnp.float32) + ab1[...])
    al_out[...] = jnp.dot(ah, aw2[...], preferred_element_type=jnp.float32) + ab2[...]


def _decoder(z, moe, gate, lib, dw1, db1, ln_g, ln_b, w_mu, b_mu, w_th, b_th,
             aw1, ab1, aw2, ab2, fw1, fb1, fw2, fb2):
    full = lambda *shape: pl.BlockSpec(shape, lambda i: (0,) * len(shape))
    row = lambda *shape: pl.BlockSpec(shape, lambda i: (i,) + (0,) * (len(shape) - 1))
    return pl.pallas_call(
        _dec_body,
        grid=(NBB,),
        in_specs=[
            row(BB, DH), row(BB, DH), row(BB, 1), row(BB, 1),
            full(DH, DH), full(DH), full(DH), full(DH),
            full(DH, NUM_GENES), full(NUM_GENES),
            full(DH, NUM_GENES), full(NUM_GENES),
            full(DH, 128), full(128), full(128, 30), full(30),
            full(DH, 64), full(64), full(64, 1), full(1),
        ],
        out_specs=[
            row(BB, NUM_GENES), row(BB, NUM_GENES), row(BB, 1), row(BB, 30),
        ],
        out_shape=[
            jax.ShapeDtypeStruct((BATCH, NUM_GENES), jnp.float32),
            jax.ShapeDtypeStruct((BATCH, NUM_GENES), jnp.float32),
            jax.ShapeDtypeStruct((BATCH, 1), jnp.float32),
            jax.ShapeDtypeStruct((BATCH, 30), jnp.float32),
        ],
        compiler_params=pltpu.CompilerParams(
            dimension_semantics=("parallel",)),
    )(z, moe, gate, lib, dw1, db1, ln_g, ln_b, w_mu, b_mu, w_th, b_th,
      aw1, ab1, aw2, ab2, fw1, fb1, fw2, fb2)


def kernel(vis, pos, grad, lib, B_f, pos_w, pos_b, img_w, img_b, router_w,
           router_b, ew1, eb1, ew2, eb2, dec_w1, dec_b1, ln_g, ln_b, dec_w2,
           dec_b2, al_w1, al_b1, al_w2, al_b2, fn_w1, fn_b1, fn_w2, fn_b2):
    z, eid, gate, cnt = _encoder(vis, pos, grad, B_f, pos_w, pos_b, img_w,
                                 img_b, router_w, router_b)
    cnt_flat = cnt.reshape(NW * NE)
    eid_flat = eid.reshape(BATCH)
    zsorted, blkexp = _dispatch(cnt_flat, eid_flat, z)
    hsorted = _moe_grouped(zsorted, blkexp, ew1.astype(jnp.bfloat16), eb1,
                           ew2.astype(jnp.bfloat16), eb2)
    moe = _unsort(cnt_flat, eid_flat, hsorted)
    w_mu = dec_w2[:, 0::2].astype(jnp.bfloat16)
    w_th = dec_w2[:, 1::2].astype(jnp.bfloat16)
    b_mu = dec_b2[0::2]
    b_th = dec_b2[1::2]
    mu, theta, func, align = _decoder(
        z, moe, gate, lib, dec_w1, dec_b1, ln_g, ln_b, w_mu, b_mu, w_th, b_th,
        al_w1, al_b1, al_w2, al_b2, fn_w1, fn_b1, fn_w2, fn_b2)
    return (mu, theta, func, align)
